# baseline (speedup 1.0000x reference)
import jax
import jax.numpy as jnp
from jax.experimental import pallas as pl
from jax.experimental.pallas import tpu as pltpu


def _xw_body(x_ref, w_ref, h_ref):
    h_ref[:] = jnp.dot(x_ref[:], w_ref[:], preferred_element_type=jnp.float32)


def _adj_body(adj_ref, h_ref, out_ref):
    out_ref[:] = jnp.dot(adj_ref[:], h_ref[:], preferred_element_type=jnp.float32)


def kernel(x, adj, w):
    n, in_dim = x.shape
    out_dim = w.shape[1]

    h = pl.pallas_call(
        _xw_body,
        out_shape=jax.ShapeDtypeStruct((n, out_dim), jnp.float32),
    )(x, w)

    bm = 400
    grid = (n // bm,)
    out = pl.pallas_call(
        _adj_body,
        grid=grid,
        in_specs=[
            pl.BlockSpec((bm, n), lambda i: (i, 0)),
            pl.BlockSpec((n, out_dim), lambda i: (0, 0)),
        ],
        out_specs=pl.BlockSpec((bm, out_dim), lambda i: (i, 0)),
        out_shape=jax.ShapeDtypeStruct((n, out_dim), jnp.float32),
        compiler_params=pltpu.CompilerParams(
            dimension_semantics=("parallel",),
        ),
    )(adj, h)
    return out


# speedup vs baseline: 1.0438x; 1.0438x over previous
import jax
import jax.numpy as jnp
from jax.experimental import pallas as pl
from jax.experimental.pallas import tpu as pltpu


def _body(adj_ref, x_ref, w_ref, out_ref):
    ax = jnp.dot(adj_ref[:], x_ref[:], preferred_element_type=jnp.float32)
    out_ref[:] = jnp.dot(ax, w_ref[:], preferred_element_type=jnp.float32)


def kernel(x, adj, w):
    n, in_dim = x.shape
    out_dim = w.shape[1]

    bm = 400
    grid = (n // bm,)
    out = pl.pallas_call(
        _body,
        grid=grid,
        in_specs=[
            pl.BlockSpec((bm, n), lambda i: (i, 0)),
            pl.BlockSpec((n, in_dim), lambda i: (0, 0)),
            pl.BlockSpec((in_dim, out_dim), lambda i: (0, 0)),
        ],
        out_specs=pl.BlockSpec((bm, out_dim), lambda i: (i, 0)),
        out_shape=jax.ShapeDtypeStruct((n, out_dim), jnp.float32),
        compiler_params=pltpu.CompilerParams(
            dimension_semantics=("parallel",),
        ),
    )(adj, x, w)
    return out
